# baseline (speedup 1.0000x reference)
import functools

import jax
import jax.numpy as jnp
from jax import lax
from jax.experimental import pallas as pl
from jax.experimental.pallas import tpu as pltpu
from jax.experimental.pallas import tpu_sc as plsc

NC = 2
NS = 16
L = 16
NW = NC * NS
B = 16384
D = 64
NV = 100000
BPW = B // NW
CHUNK = 128
NCH = BPW // CHUNK
BLKS = CHUNK // L
NCHUNK = BPW // CHUNK


def _dot_body(users_hbm, items_hbm, ue_hbm, ie_hbm, out_hbm,
              uidx, iidx, ua, ia, out_v, tsc, sem):
    wid = lax.axis_index("s") * NC + lax.axis_index("c")
    base = wid * BPW
    pltpu.sync_copy(users_hbm.at[pl.ds(wid * NCHUNK, NCHUNK)], uidx)
    pltpu.sync_copy(items_hbm.at[pl.ds(wid * NCHUNK, NCHUNK)], iidx)

    rows17 = lax.iota(jnp.int32, L) * (L + 1)

    for c in range(NCH):
        def fire(b, carry, c=c):
            uvec = uidx[c, pl.ds(b * L, L)]
            ivec = iidx[c, pl.ds(b * L, L)]
            copies = []
            for e in range(L):
                slot = b * L + e
                u = uvec[e]
                copies.append(pltpu.async_copy(
                    ue_hbm.at[u >> 3, u & 7], ua.at[slot], sem))
                it = ivec[e]
                copies.append(pltpu.async_copy(
                    ie_hbm.at[it >> 3, it & 7], ia.at[slot], sem))
            for cp in copies:
                cp.wait()
            return carry

        lax.fori_loop(0, BLKS, fire, 0)

        def blk(b, carry, c=c):
            e0 = pl.multiple_of(b * L, L)
            for e in range(L):
                s = ua[e0 + e, pl.ds(0, L)] * ia[e0 + e, pl.ds(0, L)]
                for q in range(1, D // L):
                    s = s + (ua[e0 + e, pl.ds(q * L, L)]
                             * ia[e0 + e, pl.ds(q * L, L)])
                tsc[pl.ds(e * (L + 1), L)] = s
            acc = plsc.load_gather(tsc, [rows17])
            for jj in range(1, L):
                acc = acc + plsc.load_gather(tsc, [rows17 + jj])
            out_v[pl.ds(c * CHUNK + b * L, L)] = acc
            return carry

        lax.fori_loop(0, BLKS, blk, 0)

    pltpu.sync_copy(out_v, out_hbm.at[pl.ds(base, BPW)])


def _bias_body(users_hbm, items_hbm, ub_hbm, ib_hbm, dot_hbm, out_hbm,
               uidx, iidx, ubv, ibv, dv, sem):
    wid = lax.axis_index("s") * NC + lax.axis_index("c")
    base = wid * BPW
    pltpu.sync_copy(users_hbm.at[pl.ds(wid * NCHUNK, NCHUNK)], uidx)
    pltpu.sync_copy(items_hbm.at[pl.ds(wid * NCHUNK, NCHUNK)], iidx)
    c0 = pltpu.async_copy(dot_hbm.at[pl.ds(base, BPW)], dv, sem)
    copies = [c0]
    for k in range(NCHUNK):
        copies.append(pltpu.async_copy(
            ub_hbm.at[uidx.at[k]], ubv.at[pl.ds(k * CHUNK, CHUNK)], sem))
        copies.append(pltpu.async_copy(
            ib_hbm.at[iidx.at[k]], ibv.at[pl.ds(k * CHUNK, CHUNK)], sem))
    for cp in copies:
        cp.wait()

    lanes = lax.iota(jnp.int32, L)
    zeros = lanes * 0

    def blk(b, carry):
        e0 = pl.multiple_of(b * L, L)
        ubb = plsc.load_gather(ubv, [lanes + e0, zeros])
        ibb = plsc.load_gather(ibv, [lanes + e0, zeros])
        dv[pl.ds(e0, L)] = dv[pl.ds(e0, L)] + ubb + ibb
        return carry

    lax.fori_loop(0, BPW // L, blk, 0)
    pltpu.sync_copy(dv, out_hbm.at[pl.ds(base, BPW)])


def _make_kernels():
    mesh = plsc.VectorSubcoreMesh(core_axis_name="c", subcore_axis_name="s")
    dot_kernel = functools.partial(
        pl.kernel,
        out_type=jax.ShapeDtypeStruct((B,), jnp.float32),
        mesh=mesh,
        compiler_params=pltpu.CompilerParams(needs_layout_passes=False),
        scratch_types=[
            pltpu.VMEM((NCHUNK, CHUNK), jnp.int32),
            pltpu.VMEM((NCHUNK, CHUNK), jnp.int32),
            pltpu.VMEM((CHUNK, D), jnp.float32),
            pltpu.VMEM((CHUNK, D), jnp.float32),
            pltpu.VMEM((BPW,), jnp.float32),
            pltpu.VMEM((L * (L + 1),), jnp.float32),
            pltpu.SemaphoreType.DMA,
        ],
    )(_dot_body)

    bias_kernel = functools.partial(
        pl.kernel,
        out_type=jax.ShapeDtypeStruct((B,), jnp.float32),
        mesh=mesh,
        compiler_params=pltpu.CompilerParams(
            needs_layout_passes=False, use_tc_tiling_on_sc=False),
        scratch_types=[
            pltpu.VMEM((NCHUNK, CHUNK), jnp.int32),
            pltpu.VMEM((NCHUNK, CHUNK), jnp.int32),
            pltpu.VMEM((BPW, 1), jnp.float32),
            pltpu.VMEM((BPW, 1), jnp.float32),
            pltpu.VMEM((BPW,), jnp.float32),
            pltpu.SemaphoreType.DMA,
        ],
    )(_bias_body)
    return dot_kernel, bias_kernel


_dot_kernel, _bias_kernel = None, None


def kernel(users, items, u_bias_w, i_bias_w, u_embed_w, i_embed_w):
    global _dot_kernel, _bias_kernel
    if _dot_kernel is None:
        _dot_kernel, _bias_kernel = _make_kernels()
    users2d = users.astype(jnp.int32).reshape(NW * NCHUNK, CHUNK)
    items2d = items.astype(jnp.int32).reshape(NW * NCHUNK, CHUNK)
    ue3 = u_embed_w.reshape(NV // 8, 8, D)
    ie3 = i_embed_w.reshape(NV // 8, 8, D)
    dots = _dot_kernel(users2d, items2d, ue3, ie3)
    return _bias_kernel(users2d, items2d, u_bias_w, i_bias_w, dots)


# speedup vs baseline: 1.4036x; 1.4036x over previous
import functools

import jax
import jax.numpy as jnp
from jax import lax
from jax.experimental import pallas as pl
from jax.experimental.pallas import tpu as pltpu
from jax.experimental.pallas import tpu_sc as plsc

NC = 2
NS = 16
L = 16
NW = NC * NS
B = 16384
D = 64
BPW = B // NW
CHUNK = 128
NCH = BPW // CHUNK
BLKS = CHUNK // L
NCHUNK = BPW // CHUNK


def _score_body(users_hbm, items_hbm, ue_hbm, ie_hbm, ub_hbm, ib_hbm,
                out_hbm, uidx, iidx, ua, ia, ubuf, ibuf, out_v, tsc, sem):
    wid = lax.axis_index("s") * NC + lax.axis_index("c")
    base = wid * BPW
    pltpu.sync_copy(users_hbm.at[pl.ds(wid * NCHUNK, NCHUNK)], uidx)
    pltpu.sync_copy(items_hbm.at[pl.ds(wid * NCHUNK, NCHUNK)], iidx)

    rows17 = lax.iota(jnp.int32, L) * (L + 1)
    lanes = lax.iota(jnp.int32, L)
    zeros = lanes * 0

    for c in range(NCH):
        def fire(b, carry, c=c):
            uvec = uidx[c, pl.ds(b * L, L)]
            ivec = iidx[c, pl.ds(b * L, L)]
            copies = []
            for e in range(L):
                slot = b * L + e
                u = uvec[e]
                copies.append(pltpu.async_copy(ue_hbm.at[u], ua.at[slot], sem))
                copies.append(pltpu.async_copy(
                    ub_hbm.at[u], ubuf.at[slot, pl.ds(0, 1)], sem))
                it = ivec[e]
                copies.append(pltpu.async_copy(ie_hbm.at[it], ia.at[slot], sem))
                copies.append(pltpu.async_copy(
                    ib_hbm.at[it], ibuf.at[slot, pl.ds(0, 1)], sem))
            for cp in copies:
                cp.wait()
            return carry

        lax.fori_loop(0, BLKS, fire, 0)

        def blk(b, carry, c=c):
            e0 = pl.multiple_of(b * L, L)
            for e in range(L):
                s = ua[e0 + e, pl.ds(0, L)] * ia[e0 + e, pl.ds(0, L)]
                for q in range(1, D // L):
                    s = s + (ua[e0 + e, pl.ds(q * L, L)]
                             * ia[e0 + e, pl.ds(q * L, L)])
                tsc[pl.ds(e * (L + 1), L)] = s
            acc = plsc.load_gather(tsc, [rows17])
            for jj in range(1, L):
                acc = acc + plsc.load_gather(tsc, [rows17 + jj])
            for e in range(L):
                tsc[pl.ds(e * (L + 1), L)] = (ubuf[e0 + e, pl.ds(0, L)]
                                              + ibuf[e0 + e, pl.ds(0, L)])
            acc = acc + plsc.load_gather(tsc, [rows17])
            out_v[pl.ds(c * CHUNK + b * L, L)] = acc
            return carry

        lax.fori_loop(0, BLKS, blk, 0)

    pltpu.sync_copy(out_v, out_hbm.at[pl.ds(base, BPW)])


def _make_kernel():
    mesh = plsc.VectorSubcoreMesh(core_axis_name="c", subcore_axis_name="s")
    return functools.partial(
        pl.kernel,
        out_type=jax.ShapeDtypeStruct((B,), jnp.float32),
        mesh=mesh,
        compiler_params=pltpu.CompilerParams(needs_layout_passes=False),
        scratch_types=[
            pltpu.VMEM((NCHUNK, CHUNK), jnp.int32),
            pltpu.VMEM((NCHUNK, CHUNK), jnp.int32),
            pltpu.VMEM((CHUNK, D), jnp.float32),
            pltpu.VMEM((CHUNK, D), jnp.float32),
            pltpu.VMEM((CHUNK, L), jnp.float32),
            pltpu.VMEM((CHUNK, L), jnp.float32),
            pltpu.VMEM((BPW,), jnp.float32),
            pltpu.VMEM((L * (L + 1),), jnp.float32),
            pltpu.SemaphoreType.DMA,
        ],
    )(_score_body)


_score_kernel = None


def kernel(users, items, u_bias_w, i_bias_w, u_embed_w, i_embed_w):
    global _score_kernel
    if _score_kernel is None:
        _score_kernel = _make_kernel()
    users2d = users.astype(jnp.int32).reshape(NW * NCHUNK, CHUNK)
    items2d = items.astype(jnp.int32).reshape(NW * NCHUNK, CHUNK)
    return _score_kernel(users2d, items2d, u_embed_w, i_embed_w,
                         u_bias_w, i_bias_w)


# speedup vs baseline: 1.9215x; 1.3690x over previous
import functools

import jax
import jax.numpy as jnp
from jax import lax
from jax.experimental import pallas as pl
from jax.experimental.pallas import tpu as pltpu
from jax.experimental.pallas import tpu_sc as plsc

NC = 2
NS = 16
L = 16
NW = NC * NS
B = 16384
D = 64
BPW = B // NW
CHUNK = 128
NCH = BPW // CHUNK
BLKS = CHUNK // L
NCHUNK = BPW // CHUNK


def _score_body(users_hbm, items_hbm, ue_hbm, ie_hbm, ub_hbm, ib_hbm,
                out_hbm, uidx, iidx, ua, ia, ubuf, ibuf, out_v, tsc,
                sem, sem2):
    wid = lax.axis_index("s") * NC + lax.axis_index("c")
    base = wid * BPW
    pltpu.sync_copy(users_hbm.at[pl.ds(wid * NCHUNK, NCHUNK)], uidx)
    pltpu.sync_copy(items_hbm.at[pl.ds(wid * NCHUNK, NCHUNK)], iidx)

    rows17 = lax.iota(jnp.int32, L) * (L + 1)

    bias_copies = []
    for k in range(NCHUNK):
        bias_copies.append(pltpu.async_copy(
            ub_hbm.at[uidx.at[k]], ubuf.at[pl.ds(k * CHUNK, CHUNK)], sem2))
        bias_copies.append(pltpu.async_copy(
            ib_hbm.at[iidx.at[k]], ibuf.at[pl.ds(k * CHUNK, CHUNK)], sem2))

    for c in range(NCH):
        def fire(b, carry, c=c):
            uvec = uidx[c, pl.ds(b * L, L)]
            ivec = iidx[c, pl.ds(b * L, L)]
            copies = []
            for e in range(L):
                slot = b * L + e
                u = uvec[e]
                copies.append(pltpu.async_copy(ue_hbm.at[u], ua.at[slot], sem))
                it = ivec[e]
                copies.append(pltpu.async_copy(ie_hbm.at[it], ia.at[slot], sem))
            for cp in copies:
                cp.wait()
            return carry

        lax.fori_loop(0, BLKS, fire, 0)

        def blk(b, carry, c=c):
            e0 = pl.multiple_of(b * L, L)
            for e in range(L):
                s = ua[e0 + e, pl.ds(0, L)] * ia[e0 + e, pl.ds(0, L)]
                for q in range(1, D // L):
                    s = s + (ua[e0 + e, pl.ds(q * L, L)]
                             * ia[e0 + e, pl.ds(q * L, L)])
                tsc[pl.ds(e * (L + 1), L)] = s
            acc = plsc.load_gather(tsc, [rows17])
            for jj in range(1, L):
                acc = acc + plsc.load_gather(tsc, [rows17 + jj])
            out_v[pl.ds(c * CHUNK + b * L, L)] = acc
            return carry

        lax.fori_loop(0, BLKS, blk, 0)

    for cp in bias_copies:
        cp.wait()

    def addb(b, carry):
        e0 = pl.multiple_of(b * L, L)
        out_v[pl.ds(e0, L)] = (out_v[pl.ds(e0, L)] + ubuf[pl.ds(e0, L)]
                               + ibuf[pl.ds(e0, L)])
        return carry

    lax.fori_loop(0, BPW // L, addb, 0)
    pltpu.sync_copy(out_v, out_hbm.at[pl.ds(base, BPW)])


def _make_kernel():
    mesh = plsc.VectorSubcoreMesh(core_axis_name="c", subcore_axis_name="s")
    return functools.partial(
        pl.kernel,
        out_type=jax.ShapeDtypeStruct((B,), jnp.float32),
        mesh=mesh,
        compiler_params=pltpu.CompilerParams(needs_layout_passes=False),
        scratch_types=[
            pltpu.VMEM((NCHUNK, CHUNK), jnp.int32),
            pltpu.VMEM((NCHUNK, CHUNK), jnp.int32),
            pltpu.VMEM((CHUNK, D), jnp.float32),
            pltpu.VMEM((CHUNK, D), jnp.float32),
            pltpu.VMEM((BPW,), jnp.float32),
            pltpu.VMEM((BPW,), jnp.float32),
            pltpu.VMEM((BPW,), jnp.float32),
            pltpu.VMEM((L * (L + 1),), jnp.float32),
            pltpu.SemaphoreType.DMA,
            pltpu.SemaphoreType.DMA,
        ],
    )(_score_body)


_score_kernel = None


def kernel(users, items, u_bias_w, i_bias_w, u_embed_w, i_embed_w):
    global _score_kernel
    if _score_kernel is None:
        _score_kernel = _make_kernel()
    users2d = users.astype(jnp.int32).reshape(NW * NCHUNK, CHUNK)
    items2d = items.astype(jnp.int32).reshape(NW * NCHUNK, CHUNK)
    ub_flat = u_bias_w.T.reshape(-1)
    ib_flat = i_bias_w.T.reshape(-1)
    return _score_kernel(users2d, items2d, u_embed_w, i_embed_w,
                         ub_flat, ib_flat)
